# baseline (speedup 1.0000x reference)
import functools

import jax
import jax.numpy as jnp
from jax import lax
from jax.experimental import pallas as pl
from jax.experimental.pallas import tpu as pltpu
from jax.experimental.pallas import tpu_sc as plsc

_N_BLOCK = 1024
_N_CHUNKS = 2

_NC = 2
_NS = 16
_NW = _NC * _NS
_GATHER_CHUNK = 128


def _argmin_block_kernel(zt_ref, cb2_ref, cbsq_ref, idx_ref):
    zt = zt_ref[0]
    cb2 = cb2_ref[...]
    nb = zt.shape[1]
    k = cb2.shape[0]

    scores2_t = jax.lax.dot_general(
        cb2, zt, (((1,), (0,)), ((), ())),
        preferred_element_type=jnp.float32)
    zsq = jnp.sum(zt * zt, axis=0, keepdims=True)
    d = (zsq - scores2_t) + cbsq_ref[...]

    colmin = jnp.min(d, axis=0, keepdims=True)
    sub = jax.lax.broadcasted_iota(jnp.int32, (k, nb), 0)
    idx = jnp.min(jnp.where(d == colmin, sub, k), axis=0,
                  keepdims=True)
    i = pl.program_id(0)
    idx_ref[pl.ds(i, 1), :] = idx


def _tc_argmin(zt, codebook2, cb_sq_col):
    nblocks, d_model, nb = zt.shape
    k = codebook2.shape[0]
    n = nblocks * nb
    return pl.pallas_call(
        _argmin_block_kernel,
        grid=(nblocks,),
        in_specs=[
            pl.BlockSpec((1, d_model, nb), lambda i: (i, 0, 0)),
            pl.BlockSpec((k, d_model), lambda i: (0, 0)),
            pl.BlockSpec((k, 1), lambda i: (0, 0)),
        ],
        out_specs=pl.BlockSpec((n // nb, nb), lambda i: (0, 0)),
        out_shape=jax.ShapeDtypeStruct((n // nb, nb), jnp.int32),
        compiler_params=pltpu.CompilerParams(
            dimension_semantics=("arbitrary",)),
    )(zt, codebook2, cb_sq_col)


def _sc_gather_t(codebook_t, idx_flat, b, t, d_model):
    n = b * t
    bpw = n // _NW
    tiles_per_b = t // bpw
    k = codebook_t.shape[1]
    mesh = plsc.VectorSubcoreMesh(core_axis_name="c", subcore_axis_name="s")

    @functools.partial(
        pl.kernel, mesh=mesh,
        compiler_params=pltpu.CompilerParams(use_tc_tiling_on_sc=False,
                                             needs_layout_passes=False),
        out_type=[
            jax.ShapeDtypeStruct((b, d_model, t), jnp.float32),
        ],
        scratch_types=[
            pltpu.VMEM((bpw,), jnp.int32),
            pltpu.VMEM((d_model, k), jnp.float32),
            pltpu.VMEM((d_model, bpw), jnp.float32),
            pltpu.SemaphoreType.DMA,
        ],
    )
    def sc_kernel(cbt_hbm, idx_hbm, out_a, idx_v, cbt_v, buf, sem):
        wid = lax.axis_index("s") * _NC + lax.axis_index("c")
        base = wid * bpw
        bi = wid // tiles_per_b
        t0 = (wid % tiles_per_b) * bpw
        cp_idx = pltpu.async_copy(idx_hbm.at[pl.ds(base, bpw)], idx_v, sem)
        cp_cbt = pltpu.async_copy(cbt_hbm, cbt_v, sem)
        cp_idx.wait()
        cp_cbt.wait()

        def chunk_body(c, _):
            nvec = idx_v[pl.ds(c * 16, 16)]
            for dd in range(d_model):
                dvec = jnp.full((16,), dd, jnp.int32)
                buf[dd, pl.ds(c * 16, 16)] = plsc.load_gather(
                    cbt_v, [dvec, nvec])
            return _

        lax.fori_loop(0, bpw // 16, chunk_body, None)
        pltpu.async_copy(buf, out_a.at[bi, :, pl.ds(t0, bpw)], sem).wait()

    return sc_kernel(codebook_t, idx_flat)


@jax.jit
def kernel(z, codebook):
    b, t, d_model = z.shape
    n = b * t
    zt = jnp.transpose(z, (0, 2, 1))
    codebook2 = codebook * 2.0
    cb_sq_col = jnp.sum(codebook * codebook, axis=-1)[:, None]

    cbt = jnp.transpose(codebook)
    bh = b // 2
    idx0 = _tc_argmin(zt[:bh], codebook2, cb_sq_col)
    (zq_t0,) = _sc_gather_t(cbt, idx0.reshape(bh * t), bh, t, d_model)
    idx1 = _tc_argmin(zt[bh:], codebook2, cb_sq_col)
    (zq_t1,) = _sc_gather_t(cbt, idx1.reshape(bh * t), bh, t, d_model)
    zq_t = jnp.concatenate([zq_t0, zq_t1], axis=0)
    idx = jnp.concatenate([idx0, idx1], axis=0)
    return (jnp.transpose(zq_t, (0, 2, 1)),
            jnp.transpose(zq_t, (0, 2, 1)),
            idx.reshape(b, t))


# speedup vs baseline: 1.0962x; 1.0962x over previous
import functools

import jax
import jax.numpy as jnp
from jax import lax
from jax.experimental import pallas as pl
from jax.experimental.pallas import tpu as pltpu
from jax.experimental.pallas import tpu_sc as plsc

_N_BLOCK = 1024
_N_CHUNKS = 2

_NC = 2
_NS = 16
_NW = _NC * _NS
_GATHER_CHUNK = 128


def _argmin_block_kernel(zt_ref, cb2_ref, cbsq_ref, idx_ref):
    zt = zt_ref[0]
    cb2 = cb2_ref[...]
    nb = zt.shape[1]
    k = cb2.shape[0]

    scores2_t = jax.lax.dot_general(
        cb2, zt, (((1,), (0,)), ((), ())),
        preferred_element_type=jnp.float32)
    zsq = jnp.sum(zt * zt, axis=0, keepdims=True)
    d = (zsq - scores2_t) + cbsq_ref[...]

    colmin = jnp.min(d, axis=0, keepdims=True)
    sub = jax.lax.broadcasted_iota(jnp.int32, (k, nb), 0)
    idx = jnp.min(jnp.where(d == colmin, sub, k), axis=0,
                  keepdims=True)
    i = pl.program_id(0)
    idx_ref[pl.ds(i, 1), :] = idx


def _tc_argmin(zt, codebook2, cb_sq_col):
    nblocks, d_model, nb = zt.shape
    k = codebook2.shape[0]
    n = nblocks * nb
    return pl.pallas_call(
        _argmin_block_kernel,
        grid=(nblocks,),
        in_specs=[
            pl.BlockSpec((1, d_model, nb), lambda i: (i, 0, 0)),
            pl.BlockSpec((k, d_model), lambda i: (0, 0)),
            pl.BlockSpec((k, 1), lambda i: (0, 0)),
        ],
        out_specs=pl.BlockSpec((n // nb, nb), lambda i: (0, 0)),
        out_shape=jax.ShapeDtypeStruct((n // nb, nb), jnp.int32),
        compiler_params=pltpu.CompilerParams(
            dimension_semantics=("arbitrary",)),
    )(zt, codebook2, cb_sq_col)


def _sc_gather_t(codebook_t, idx_flat, b, t, d_model):
    n = b * t
    bpw = n // _NW
    tiles_per_b = t // bpw
    k = codebook_t.shape[1]
    mesh = plsc.VectorSubcoreMesh(core_axis_name="c", subcore_axis_name="s")

    @functools.partial(
        pl.kernel, mesh=mesh,
        compiler_params=pltpu.CompilerParams(use_tc_tiling_on_sc=False,
                                             needs_layout_passes=False),
        out_type=[
            jax.ShapeDtypeStruct((b, d_model, t), jnp.float32),
        ],
        scratch_types=[
            pltpu.VMEM((bpw,), jnp.int32),
            pltpu.VMEM((d_model, k), jnp.float32),
            pltpu.VMEM((d_model, bpw), jnp.float32),
            pltpu.SemaphoreType.DMA,
        ],
    )
    def sc_kernel(cbt_hbm, idx_hbm, out_a, idx_v, cbt_v, buf, sem):
        wid = lax.axis_index("s") * _NC + lax.axis_index("c")
        base = wid * bpw
        bi = wid // tiles_per_b
        t0 = (wid % tiles_per_b) * bpw
        cp_idx = pltpu.async_copy(idx_hbm.at[pl.ds(base, bpw)], idx_v, sem)
        cp_cbt = pltpu.async_copy(cbt_hbm, cbt_v, sem)
        cp_idx.wait()
        cp_cbt.wait()

        def chunk_body(c, _):
            nvec = idx_v[pl.ds(c * 16, 16)]
            for dd in range(d_model):
                dvec = jnp.full((16,), dd, jnp.int32)
                buf[dd, pl.ds(c * 16, 16)] = plsc.load_gather(
                    cbt_v, [dvec, nvec])
            return _

        lax.fori_loop(0, bpw // 16, chunk_body, None)
        pltpu.async_copy(buf, out_a.at[bi, :, pl.ds(t0, bpw)], sem).wait()

    return sc_kernel(codebook_t, idx_flat)


@jax.jit
def kernel(z, codebook):
    b, t, d_model = z.shape
    n = b * t
    zt = jnp.transpose(z, (0, 2, 1))
    codebook2 = codebook * 2.0
    cb_sq_col = jnp.sum(codebook * codebook, axis=-1)[:, None]

    idx = _tc_argmin(zt, codebook2, cb_sq_col)
    (zq_t,) = _sc_gather_t(jnp.transpose(codebook),
                           idx.reshape(n), b, t, d_model)
    return (jnp.transpose(zq_t, (0, 2, 1)),
            jnp.transpose(zq_t, (0, 2, 1)),
            idx.reshape(b, t))
